# Initial kernel scaffold; baseline (speedup 1.0000x reference)
import jax
import jax.numpy as jnp
from jax.experimental import pallas as pl


def kernel(updates, mask):
    raise NotImplementedError("write your pallas kernel here")


# speedup vs baseline: 3.9013x; 3.9013x over previous
import functools

import jax
import jax.numpy as jnp
from jax import lax
from jax.experimental import pallas as pl
from jax.experimental.pallas import tpu as pltpu
from jax.experimental.pallas import tpu_sc as plsc

_B, _H, _W, _C = 4, 192, 192, 96
_HO, _WO = 2 * _H, 2 * _W
_INB = _H * _W * _C
_OUTB = _HO * _WO * _C
_OUT = _B * _OUTB

_NC, _NS, _L = 2, 16, 16
_BPC = _B // _NC
_SH = _BPC * _INB // _NS
_S = 16
_CH = _BPC * _OUTB // _S
_K = 4608
_NBLK = _SH // _K
_NG = _K // _L
_ZS = _CH // _NS
_NZ = _ZS // _K
_WPB = _INB // _SH

_mesh = plsc.VectorSubcoreMesh(
    core_axis_name="c", subcore_axis_name="s",
    num_cores=_NC, num_subcores=_NS)


@functools.partial(
    pl.kernel,
    out_type=jax.ShapeDtypeStruct((_OUT,), jnp.float32),
    mesh=_mesh,
    scratch_types=[
        pltpu.VMEM((_K,), jnp.int32),
        pltpu.VMEM((_K,), jnp.float32),
        pltpu.VMEM((_K,), jnp.int32),
        pltpu.VMEM((_K,), jnp.float32),
        pltpu.VMEM_SHARED((_CH,), jnp.float32),
    ],
)
def _unpool(mask_hbm, upd_hbm, out_hbm, mbuf, ubuf, idxbuf, vbuf, acc):
  ci = lax.axis_index("c")
  si = lax.axis_index("s")

  in_base = ci * (_BPC * _INB) + si * _SH
  out_base = ci * (_BPC * _OUTB)
  gbase = (ci * _BPC + si // _WPB) * _OUTB

  for s in range(_S):
    lo = out_base + s * _CH

    def zfill(i, carry):
      vbuf[pl.ds(i * _L, _L)] = jnp.zeros((_L,), jnp.float32)
      return carry
    lax.fori_loop(0, _NG, zfill, 0)
    for t in range(_NZ):
      pltpu.sync_copy(vbuf, acc.at[pl.ds(si * _ZS + t * _K, _K)])
    plsc.subcore_barrier()

    def blk_body(blk, carry):
      base = in_base + blk * _K
      pltpu.sync_copy(mask_hbm.at[pl.ds(base, _K)], mbuf)
      pltpu.sync_copy(upd_hbm.at[pl.ds(base, _K)], ubuf)

      def grp(j, c):
        m = mbuf[pl.ds(j * _L, _L)]
        v = ubuf[pl.ds(j * _L, _L)]
        cv = jnp.full((_L,), _C, jnp.int32)
        chv = jnp.full((_L,), _CH, jnp.int32)
        zv = jnp.full((_L,), 0, jnp.int32)
        nv = jnp.full((_L,), -1, jnp.int32)
        c0 = (j % 6) * _L
        third = jnp.full((_L,), jnp.float32(1.0 / 3.0))
        q = (lax.shift_right_logical(m, jnp.full((_L,), 5, jnp.int32))
             .astype(jnp.float32) * third).astype(jnp.int32)
        off = jnp.full((_L,), c0 + gbase - lo, jnp.int32) + lax.iota(
            jnp.int32, _L)
        idx = q * cv + off
        ok = (idx >= zv) & (idx < chv)
        idxbuf[pl.ds(j * _L, _L)] = jnp.where(ok, idx, nv)
        vbuf[pl.ds(j * _L, _L)] = v
        return c
      lax.fori_loop(0, _NG, grp, 0)

      pltpu.sync_copy(
          vbuf, acc.at[plsc.Indices(idxbuf, ignored_value=-1)], add=True)
      return carry
    lax.fori_loop(0, _NBLK, blk_body, 0)
    plsc.subcore_barrier()

    pltpu.sync_copy(acc.at[pl.ds(si * _ZS, _ZS)],
                    out_hbm.at[pl.ds(lo + si * _ZS, _ZS)])
    plsc.subcore_barrier()


def kernel(updates, mask):
  m = mask.astype(jnp.int32).reshape(-1)
  u = updates.reshape(-1)
  out = _unpool(m, u)
  return out.reshape(_B, _HO, _WO, _C)


# speedup vs baseline: 6.8727x; 1.7617x over previous
import functools

import jax
import jax.numpy as jnp
from jax import lax
from jax.experimental import pallas as pl
from jax.experimental.pallas import tpu as pltpu
from jax.experimental.pallas import tpu_sc as plsc

_B, _H, _W, _C = 4, 192, 192, 96
_HO, _WO = 2 * _H, 2 * _W
_INB = _H * _W * _C
_OUTB = _HO * _WO * _C
_OUT = _B * _OUTB

_NC, _NS, _L = 2, 16, 16
_BPC = _B // _NC
_SH = _BPC * _INB // _NS
_S = 16
_CH = _BPC * _OUTB // _S
_K = 1728
_NBLK = _SH // _K
_NB2 = _NBLK // 2
_NG = _K // _L
_ZS = _CH // _NS
_NZ = _ZS // _K
_WPB = _INB // _SH

_mesh = plsc.VectorSubcoreMesh(
    core_axis_name="c", subcore_axis_name="s",
    num_cores=_NC, num_subcores=_NS)


@functools.partial(
    pl.kernel,
    out_type=jax.ShapeDtypeStruct((_OUT,), jnp.float32),
    mesh=_mesh,
    scratch_types=[
        pltpu.VMEM((_K,), jnp.int32),
        pltpu.VMEM((_K,), jnp.float32),
        pltpu.VMEM((_K,), jnp.int32),
        pltpu.VMEM((_K,), jnp.float32),
        pltpu.VMEM((_K,), jnp.int32),
        pltpu.VMEM((_K,), jnp.float32),
        pltpu.VMEM((_K,), jnp.int32),
        pltpu.VMEM((_K,), jnp.float32),
        pltpu.VMEM((_K,), jnp.float32),
        pltpu.VMEM_SHARED((_CH,), jnp.float32),
        pltpu.SemaphoreType.DMA,
        pltpu.SemaphoreType.DMA,
        pltpu.SemaphoreType.DMA,
        pltpu.SemaphoreType.DMA,
        pltpu.SemaphoreType.DMA,
    ],
)
def _unpool(mask_hbm, upd_hbm, out_hbm,
            mba, uba, iba, vba, mbb, ubb, ibb, vbb, zbuf, acc,
            sin_a, sin_b, ssc_a, ssc_b, sz):
  ci = lax.axis_index("c")
  si = lax.axis_index("s")

  def zinit(i, carry):
    zbuf[pl.ds(i * _L, _L)] = jnp.zeros((_L,), jnp.float32)
    return carry
  lax.fori_loop(0, _NG, zinit, 0)

  in_base = ci * (_BPC * _INB) + si * _SH
  out_base = ci * (_BPC * _OUTB)
  gbase = (ci * _BPC + si // _WPB) * _OUTB

  def start_in(blk, mb, ub, sem):
    base = in_base + blk * _K
    pltpu.async_copy(mask_hbm.at[pl.ds(base, _K)], mb, sem)
    pltpu.async_copy(upd_hbm.at[pl.ds(base, _K)], ub, sem)

  def wait_in(blk, mb, ub, sem):
    base = in_base + blk * _K
    pltpu.make_async_copy(mask_hbm.at[pl.ds(base, _K)], mb, sem).wait()
    pltpu.make_async_copy(upd_hbm.at[pl.ds(base, _K)], ub, sem).wait()

  def decode(mb, ub, ib, vb, lo):
    def grp(j, c):
      m = mb[pl.ds(j * _L, _L)]
      vb[pl.ds(j * _L, _L)] = ub[pl.ds(j * _L, _L)]
      cv = jnp.full((_L,), _C, jnp.int32)
      chv = jnp.full((_L,), _CH, jnp.int32)
      zv = jnp.full((_L,), 0, jnp.int32)
      nv = jnp.full((_L,), -1, jnp.int32)
      c0 = (j % 6) * _L
      third = jnp.full((_L,), jnp.float32(1.0 / 3.0))
      q = (lax.shift_right_logical(m, jnp.full((_L,), 5, jnp.int32))
           .astype(jnp.float32) * third).astype(jnp.int32)
      off = jnp.full((_L,), c0 + gbase - lo, jnp.int32) + lax.iota(
          jnp.int32, _L)
      idx = q * cv + off
      ok = (idx >= zv) & (idx < chv)
      ib[pl.ds(j * _L, _L)] = jnp.where(ok, idx, nv)
      return c
    lax.fori_loop(0, _NG, grp, 0)

  def start_sc(ub, ib, sem):
    pltpu.async_copy(
        ub, acc.at[plsc.Indices(ib, ignored_value=-1)], sem, add=True)

  def wait_sc(ub, ib, sem):
    pltpu.make_async_copy(
        ub, acc.at[plsc.Indices(ib, ignored_value=-1)], sem).wait()

  for s in range(_S):
    lo = out_base + s * _CH

    for t in range(_NZ):
      pltpu.async_copy(zbuf, acc.at[pl.ds(si * _ZS + t * _K, _K)], sz)
    for t in range(_NZ):
      pltpu.make_async_copy(
          zbuf, acc.at[pl.ds(si * _ZS + t * _K, _K)], sz).wait()
    plsc.subcore_barrier()

    start_in(0, mba, uba, sin_a)
    start_in(1, mbb, ubb, sin_b)

    def pipe(i, carry):
      wait_in(2 * i, mba, uba, sin_a)

      @pl.when(i > 0)
      def _():
        wait_sc(vba, iba, ssc_a)
      decode(mba, uba, iba, vba, lo)
      start_sc(vba, iba, ssc_a)

      @pl.when(i < _NB2 - 1)
      def _():
        start_in(2 * i + 2, mba, uba, sin_a)

      wait_in(2 * i + 1, mbb, ubb, sin_b)

      @pl.when(i > 0)
      def _():
        wait_sc(vbb, ibb, ssc_b)
      decode(mbb, ubb, ibb, vbb, lo)
      start_sc(vbb, ibb, ssc_b)

      @pl.when(i < _NB2 - 1)
      def _():
        start_in(2 * i + 3, mbb, ubb, sin_b)
      return carry
    lax.fori_loop(0, _NB2, pipe, 0)
    wait_sc(vba, iba, ssc_a)
    wait_sc(vbb, ibb, ssc_b)
    plsc.subcore_barrier()

    pltpu.sync_copy(acc.at[pl.ds(si * _ZS, _ZS)],
                    out_hbm.at[pl.ds(lo + si * _ZS, _ZS)])
    plsc.subcore_barrier()


def kernel(updates, mask):
  m = mask.astype(jnp.int32).reshape(-1)
  u = updates.reshape(-1)
  out = _unpool(m, u)
  return out.reshape(_B, _HO, _WO, _C)


# speedup vs baseline: 10.0430x; 1.4613x over previous
import functools

import jax
import jax.numpy as jnp
from jax import lax
from jax.experimental import pallas as pl
from jax.experimental.pallas import tpu as pltpu
from jax.experimental.pallas import tpu_sc as plsc

_B, _H, _W, _C = 4, 192, 192, 96
_HO, _WO = 2 * _H, 2 * _W
_INB = _H * _W * _C
_OUTB = _HO * _WO * _C
_OUT = _B * _OUTB

_NC, _NS, _L = 2, 16, 16
_BPC = _B // _NC
_SH = _BPC * _INB // _NS
_S = 16
_CH = _BPC * _OUTB // _S
_K = 1728
_NBLK = _SH // _K
_NB2 = _NBLK // 2
_NG = _K // _L
_NCH = _NG // 6
_PB = 1728
_PBW = _PB + 96
_PGR = _PBW // _L
_ZS = _CH // _NS
_NZ = _ZS // _K
_WPB = _INB // _SH

_mesh = plsc.VectorSubcoreMesh(
    core_axis_name="c", subcore_axis_name="s",
    num_cores=_NC, num_subcores=_NS)


@functools.partial(
    pl.kernel,
    out_type=jax.ShapeDtypeStruct((_OUT,), jnp.float32),
    mesh=_mesh,
    compiler_params=pltpu.CompilerParams(needs_layout_passes=False),
    scratch_types=[
        pltpu.VMEM((_K,), jnp.int32),
        pltpu.VMEM((_K,), jnp.float32),
        pltpu.VMEM((_K,), jnp.int32),
        pltpu.VMEM((_K,), jnp.float32),
        pltpu.VMEM((_PBW,), jnp.int32),
        pltpu.VMEM((_PBW,), jnp.float32),
        pltpu.VMEM((_K,), jnp.float32),
        pltpu.VMEM_SHARED((_CH,), jnp.float32),
        pltpu.SMEM((8,), jnp.int32),
        pltpu.SemaphoreType.DMA,
        pltpu.SemaphoreType.DMA,
        pltpu.SemaphoreType.DMA,
    ],
)
def _unpool(mask_hbm, upd_hbm, out_hbm,
            mba, uba, mbb, ubb, pidx, pval, zbuf, acc, pos_ref,
            sin_a, sin_b, sz):
  ci = lax.axis_index("c")
  si = lax.axis_index("s")

  def zinit(i, carry):
    zbuf[pl.ds(i * _L, _L)] = jnp.zeros((_L,), jnp.float32)
    return carry
  lax.fori_loop(0, _K // _L, zinit, 0)

  in_base = ci * (_BPC * _INB) + si * _SH
  out_base = ci * (_BPC * _OUTB)
  gbase = (ci * _BPC + si // _WPB) * _OUTB

  def start_in(blk, mb, ub, sem):
    base = in_base + blk * _K
    pltpu.async_copy(mask_hbm.at[pl.ds(base, _K)], mb, sem)
    pltpu.async_copy(upd_hbm.at[pl.ds(base, _K)], ub, sem)

  def wait_in(blk, mb, ub, sem):
    base = in_base + blk * _K
    pltpu.make_async_copy(mask_hbm.at[pl.ds(base, _K)], mb, sem).wait()
    pltpu.make_async_copy(upd_hbm.at[pl.ds(base, _K)], ub, sem).wait()

  def flush():
    pos = pos_ref[0]

    def clr(t, carry):
      old = pidx[pl.ds(t * _L, _L)]
      keep = (jnp.full((_L,), t * _L, jnp.int32) + lax.iota(jnp.int32, _L)
              ) < jnp.full((_L,), pos, jnp.int32)
      pidx[pl.ds(t * _L, _L)] = jnp.where(
          keep, old, jnp.full((_L,), -1, jnp.int32))
      return carry
    lax.fori_loop(pos // _L, _PGR, clr, 0)
    pltpu.sync_copy(
        pval, acc.at[plsc.Indices(pidx, ignored_value=-1)], add=True)
    pos_ref[0] = 0

  def consume(mb, ub, lo):
    def chunk(t, carry):
      lis, vs, mks, cnts = [], [], [], []
      for u in range(6):
        j = t * 6 + u
        m = mb[pl.ds(j * _L, _L)]
        v = ub[pl.ds(j * _L, _L)]
        cv = jnp.full((_L,), _C, jnp.int32)
        third = jnp.full((_L,), jnp.float32(1.0 / 3.0))
        q = (lax.shift_right_logical(m, jnp.full((_L,), 5, jnp.int32))
             .astype(jnp.float32) * third).astype(jnp.int32)
        off = jnp.full((_L,), u * _L + gbase - lo, jnp.int32) + lax.iota(
            jnp.int32, _L)
        li = q * cv + off
        mk = (li >= jnp.full((_L,), 0, jnp.int32)) & (
            li < jnp.full((_L,), _CH, jnp.int32))
        lis.append(li)
        vs.append(v)
        mks.append(mk)
        ones = jnp.where(mk, jnp.full((_L,), 1, jnp.int32),
                         jnp.full((_L,), 0, jnp.int32))
        cnts.append(jnp.sum(ones))
      base = pos_ref[0]
      for u in range(6):
        plsc.store_compressed(pidx.at[pl.ds(base, _L)], lis[u], mask=mks[u])
        plsc.store_compressed(pval.at[pl.ds(base, _L)], vs[u], mask=mks[u])
        base = base + cnts[u]
      pos_ref[0] = base

      @pl.when(base >= _PB)
      def _():
        flush()
      return carry
    lax.fori_loop(0, _NCH, chunk, 0)

  def sweep(s, scarry):
    lo = out_base + s * _CH

    def zstart(t, carry):
      pltpu.async_copy(zbuf, acc.at[pl.ds(si * _ZS + t * _K, _K)], sz)
      return carry
    lax.fori_loop(0, _NZ, zstart, 0)

    def zwait(t, carry):
      pltpu.make_async_copy(
          zbuf, acc.at[pl.ds(si * _ZS + t * _K, _K)], sz).wait()
      return carry
    lax.fori_loop(0, _NZ, zwait, 0)
    plsc.subcore_barrier()

    pos_ref[0] = 0
    start_in(0, mba, uba, sin_a)
    start_in(1, mbb, ubb, sin_b)

    def pipe(i, carry):
      wait_in(2 * i, mba, uba, sin_a)
      consume(mba, uba, lo)

      @pl.when(i < _NB2 - 1)
      def _():
        start_in(2 * i + 2, mba, uba, sin_a)

      wait_in(2 * i + 1, mbb, ubb, sin_b)
      consume(mbb, ubb, lo)

      @pl.when(i < _NB2 - 1)
      def _():
        start_in(2 * i + 3, mbb, ubb, sin_b)
      return carry
    lax.fori_loop(0, _NB2, pipe, 0)
    flush()
    plsc.subcore_barrier()

    pltpu.sync_copy(acc.at[pl.ds(si * _ZS, _ZS)],
                    out_hbm.at[pl.ds(lo + si * _ZS, _ZS)])
    plsc.subcore_barrier()
    return scarry

  lax.fori_loop(0, _S, sweep, 0)


def kernel(updates, mask):
  m = mask.astype(jnp.int32).reshape(-1)
  u = updates.reshape(-1)
  out = _unpool(m, u)
  return out.reshape(_B, _HO, _WO, _C)


# speedup vs baseline: 10.4029x; 1.0358x over previous
import functools

import jax
import jax.numpy as jnp
from jax import lax
from jax.experimental import pallas as pl
from jax.experimental.pallas import tpu as pltpu
from jax.experimental.pallas import tpu_sc as plsc

_B, _H, _W, _C = 4, 192, 192, 96
_HO, _WO = 2 * _H, 2 * _W
_INB = _H * _W * _C
_OUTB = _HO * _WO * _C
_OUT = _B * _OUTB

_NC, _NS, _L = 2, 16, 16
_BPC = _B // _NC
_SH = _BPC * _INB // _NS
_S = 16
_CH = _BPC * _OUTB // _S
_K = 1728
_NBLK = _SH // _K
_NB2 = _NBLK // 2
_NG = _K // _L
_NCH = _NG // 6
_PB = 1728
_PBW = _PB + 96
_PGR = _PBW // _L
_ZS = _CH // _NS
_NZ = _ZS // _K
_WPB = _INB // _SH

_mesh = plsc.VectorSubcoreMesh(
    core_axis_name="c", subcore_axis_name="s",
    num_cores=_NC, num_subcores=_NS)


@functools.partial(
    pl.kernel,
    out_type=jax.ShapeDtypeStruct((_OUT,), jnp.float32),
    mesh=_mesh,
    compiler_params=pltpu.CompilerParams(needs_layout_passes=False),
    scratch_types=[
        pltpu.VMEM((_K,), jnp.int32),
        pltpu.VMEM((_K,), jnp.float32),
        pltpu.VMEM((_K,), jnp.int32),
        pltpu.VMEM((_K,), jnp.float32),
        pltpu.VMEM((_PBW,), jnp.int32),
        pltpu.VMEM((_PBW,), jnp.float32),
        pltpu.VMEM((_K,), jnp.float32),
        pltpu.VMEM_SHARED((_CH,), jnp.float32),
        pltpu.SMEM((8,), jnp.int32),
        pltpu.SemaphoreType.DMA,
        pltpu.SemaphoreType.DMA,
        pltpu.SemaphoreType.DMA,
    ],
)
def _unpool(mask_hbm, upd_hbm, out_hbm,
            mba, uba, mbb, ubb, pidx, pval, zbuf, acc, pos_ref,
            sin_a, sin_b, sz):
  ci = lax.axis_index("c")
  si = lax.axis_index("s")

  def zinit(i, carry):
    zbuf[pl.ds(i * _L, _L)] = jnp.zeros((_L,), jnp.float32)
    return carry
  lax.fori_loop(0, _K // _L, zinit, 0)

  in_base = ci * (_BPC * _INB) + si * _SH
  out_base = ci * (_BPC * _OUTB)
  gbase = (ci * _BPC + si // _WPB) * _OUTB

  def start_in(blk, mb, ub, sem):
    base = in_base + blk * _K
    pltpu.async_copy(mask_hbm.at[pl.ds(base, _K)], mb, sem)
    pltpu.async_copy(upd_hbm.at[pl.ds(base, _K)], ub, sem)

  def wait_in(blk, mb, ub, sem):
    base = in_base + blk * _K
    pltpu.make_async_copy(mask_hbm.at[pl.ds(base, _K)], mb, sem).wait()
    pltpu.make_async_copy(upd_hbm.at[pl.ds(base, _K)], ub, sem).wait()

  def flush():
    pos = pos_ref[0]

    def clr(t, carry):
      old = pidx[pl.ds(t * _L, _L)]
      keep = (jnp.full((_L,), t * _L, jnp.int32) + lax.iota(jnp.int32, _L)
              ) < jnp.full((_L,), pos, jnp.int32)
      pidx[pl.ds(t * _L, _L)] = jnp.where(
          keep, old, jnp.full((_L,), -1, jnp.int32))
      return carry
    lax.fori_loop(pos // _L, _PGR, clr, 0)
    pltpu.sync_copy(
        pval, acc.at[plsc.Indices(pidx, ignored_value=-1)], add=True)
    pos_ref[0] = 0

  def consume(mb, ub, lo):
    def chunk(t, carry):
      lis, vs, mks, cnts = [], [], [], []
      for u in range(6):
        j = t * 6 + u
        m = mb[pl.ds(j * _L, _L)]
        v = ub[pl.ds(j * _L, _L)]
        cv = jnp.full((_L,), _C, jnp.int32)
        third = jnp.full((_L,), jnp.float32(1.0 / 3.0))
        q = (lax.shift_right_logical(m, jnp.full((_L,), 5, jnp.int32))
             .astype(jnp.float32) * third).astype(jnp.int32)
        off = jnp.full((_L,), u * _L + gbase - lo, jnp.int32) + lax.iota(
            jnp.int32, _L)
        li = q * cv + off
        mk = (li >= jnp.full((_L,), 0, jnp.int32)) & (
            li < jnp.full((_L,), _CH, jnp.int32))
        lis.append(li)
        vs.append(v)
        mks.append(mk)
        cnts.append(plsc.all_reduce_population_count(mk)[0])
      base = pos_ref[0]
      for u in range(6):
        plsc.store_compressed(pidx.at[pl.ds(base, _L)], lis[u], mask=mks[u])
        plsc.store_compressed(pval.at[pl.ds(base, _L)], vs[u], mask=mks[u])
        base = base + cnts[u]
      pos_ref[0] = base

      @pl.when(base >= _PB)
      def _():
        flush()
      return carry
    lax.fori_loop(0, _NCH, chunk, 0)

  def sweep(s, scarry):
    lo = out_base + s * _CH

    def zstart(t, carry):
      pltpu.async_copy(zbuf, acc.at[pl.ds(si * _ZS + t * _K, _K)], sz)
      return carry
    lax.fori_loop(0, _NZ, zstart, 0)

    def zwait(t, carry):
      pltpu.make_async_copy(
          zbuf, acc.at[pl.ds(si * _ZS + t * _K, _K)], sz).wait()
      return carry
    lax.fori_loop(0, _NZ, zwait, 0)
    plsc.subcore_barrier()

    pos_ref[0] = 0
    start_in(0, mba, uba, sin_a)
    start_in(1, mbb, ubb, sin_b)

    def pipe(i, carry):
      wait_in(2 * i, mba, uba, sin_a)
      consume(mba, uba, lo)

      @pl.when(i < _NB2 - 1)
      def _():
        start_in(2 * i + 2, mba, uba, sin_a)

      wait_in(2 * i + 1, mbb, ubb, sin_b)
      consume(mbb, ubb, lo)

      @pl.when(i < _NB2 - 1)
      def _():
        start_in(2 * i + 3, mbb, ubb, sin_b)
      return carry
    lax.fori_loop(0, _NB2, pipe, 0)
    flush()
    plsc.subcore_barrier()

    pltpu.sync_copy(acc.at[pl.ds(si * _ZS, _ZS)],
                    out_hbm.at[pl.ds(lo + si * _ZS, _ZS)])
    plsc.subcore_barrier()
    return scarry

  lax.fori_loop(0, _S, sweep, 0)


def kernel(updates, mask):
  m = mask.astype(jnp.int32).reshape(-1)
  u = updates.reshape(-1)
  out = _unpool(m, u)
  return out.reshape(_B, _HO, _WO, _C)


# speedup vs baseline: 11.7854x; 1.1329x over previous
import functools

import jax
import jax.numpy as jnp
from jax import lax
from jax.experimental import pallas as pl
from jax.experimental.pallas import tpu as pltpu
from jax.experimental.pallas import tpu_sc as plsc

_B, _H, _W, _C = 4, 192, 192, 96
_HO, _WO = 2 * _H, 2 * _W
_INB = _H * _W * _C
_OUTB = _HO * _WO * _C
_OUT = _B * _OUTB

_NC, _NS, _L = 2, 16, 16
_BPC = _B // _NC
_SH = _BPC * _INB // _NS
_S = 16
_CH = _BPC * _OUTB // _S
_K = 1728
_NBLK = _SH // _K
_NB2 = _NBLK // 2
_NG = _K // _L
_GPC = 12
_NCH = _NG // _GPC
_PB = 1728
_PBW = _PB + _GPC * _L
_PGR = _PBW // _L
_ZS = _CH // _NS
_NZ = _ZS // _K
_WPB = _INB // _SH

_mesh = plsc.VectorSubcoreMesh(
    core_axis_name="c", subcore_axis_name="s",
    num_cores=_NC, num_subcores=_NS)


@functools.partial(
    pl.kernel,
    out_type=jax.ShapeDtypeStruct((_OUT,), jnp.float32),
    mesh=_mesh,
    compiler_params=pltpu.CompilerParams(needs_layout_passes=False),
    scratch_types=[
        pltpu.VMEM((_K,), jnp.int32),
        pltpu.VMEM((_K,), jnp.float32),
        pltpu.VMEM((_K,), jnp.int32),
        pltpu.VMEM((_K,), jnp.float32),
        pltpu.VMEM((_PBW,), jnp.int32),
        pltpu.VMEM((_PBW,), jnp.float32),
        pltpu.VMEM((_K,), jnp.float32),
        pltpu.VMEM_SHARED((_CH,), jnp.float32),
        pltpu.SemaphoreType.DMA,
        pltpu.SemaphoreType.DMA,
        pltpu.SemaphoreType.DMA,
    ],
)
def _unpool(mask_hbm, upd_hbm, out_hbm,
            mba, uba, mbb, ubb, pidx, pval, zbuf, acc,
            sin_a, sin_b, sz):
  ci = lax.axis_index("c")
  si = lax.axis_index("s")

  def zinit(i, carry):
    zbuf[pl.ds(i * _L, _L)] = jnp.zeros((_L,), jnp.float32)
    return carry
  lax.fori_loop(0, _K // _L, zinit, 0)

  in_base = ci * (_BPC * _INB) + si * _SH
  out_base = ci * (_BPC * _OUTB)
  gbase = (ci * _BPC + si // _WPB) * _OUTB

  def start_in(blk, mb, ub, sem):
    base = in_base + blk * _K
    pltpu.async_copy(mask_hbm.at[pl.ds(base, _K)], mb, sem)
    pltpu.async_copy(upd_hbm.at[pl.ds(base, _K)], ub, sem)

  def wait_in(blk, mb, ub, sem):
    base = in_base + blk * _K
    pltpu.make_async_copy(mask_hbm.at[pl.ds(base, _K)], mb, sem).wait()
    pltpu.make_async_copy(upd_hbm.at[pl.ds(base, _K)], ub, sem).wait()

  def flush(pos):
    def clr(t, carry):
      old = pidx[pl.ds(t * _L, _L)]
      keep = (jnp.full((_L,), t * _L, jnp.int32) + lax.iota(jnp.int32, _L)
              ) < jnp.full((_L,), pos, jnp.int32)
      pidx[pl.ds(t * _L, _L)] = jnp.where(
          keep, old, jnp.full((_L,), -1, jnp.int32))
      return carry
    lax.fori_loop(pos // _L, _PGR, clr, 0)
    pltpu.sync_copy(
        pval, acc.at[plsc.Indices(pidx, ignored_value=-1)], add=True)
    return jnp.int32(0)

  def consume(mb, ub, lo, pos):
    chv = jnp.full((_L,), _CH, jnp.uint32)

    def chunk(t, pos):
      lis, vs, cnts = [], [], []
      for u in range(_GPC):
        j = t * _GPC + u
        m = mb[pl.ds(j * _L, _L)]
        v = ub[pl.ds(j * _L, _L)]
        cv = jnp.full((_L,), _C, jnp.int32)
        third = jnp.full((_L,), jnp.float32(1.0 / 3.0))
        q = (lax.shift_right_logical(m, jnp.full((_L,), 5, jnp.int32))
             .astype(jnp.float32) * third).astype(jnp.int32)
        off = jnp.full((_L,), (u % 6) * _L + gbase - lo,
                       jnp.int32) + lax.iota(jnp.int32, _L)
        li = q * cv + off
        mk = plsc.bitcast(li, jnp.uint32) < chv
        lis.append(li)
        vs.append(v)
        cnts.append(plsc.all_reduce_population_count(mk)[0])
      base = pos
      for u in range(_GPC):
        mk = plsc.bitcast(lis[u], jnp.uint32) < chv
        plsc.store_compressed(pidx.at[pl.ds(base, _L)], lis[u], mask=mk)
        plsc.store_compressed(pval.at[pl.ds(base, _L)], vs[u], mask=mk)
        base = base + cnts[u]
      return lax.cond(base >= _PB, flush, lambda p: p, base)
    return lax.fori_loop(0, _NCH, chunk, pos)

  def sweep(s, scarry):
    lo = out_base + s * _CH

    def zstart(t, carry):
      pltpu.async_copy(zbuf, acc.at[pl.ds(si * _ZS + t * _K, _K)], sz)
      return carry
    lax.fori_loop(0, _NZ, zstart, 0)

    def zwait(t, carry):
      pltpu.make_async_copy(
          zbuf, acc.at[pl.ds(si * _ZS + t * _K, _K)], sz).wait()
      return carry
    lax.fori_loop(0, _NZ, zwait, 0)
    plsc.subcore_barrier()

    start_in(0, mba, uba, sin_a)
    start_in(1, mbb, ubb, sin_b)

    def pipe(i, pos):
      wait_in(2 * i, mba, uba, sin_a)
      pos = consume(mba, uba, lo, pos)

      @pl.when(i < _NB2 - 1)
      def _():
        start_in(2 * i + 2, mba, uba, sin_a)

      wait_in(2 * i + 1, mbb, ubb, sin_b)
      pos = consume(mbb, ubb, lo, pos)

      @pl.when(i < _NB2 - 1)
      def _():
        start_in(2 * i + 3, mbb, ubb, sin_b)
      return pos
    pos = lax.fori_loop(0, _NB2, pipe, jnp.int32(0))
    flush(pos)
    plsc.subcore_barrier()

    pltpu.sync_copy(acc.at[pl.ds(si * _ZS, _ZS)],
                    out_hbm.at[pl.ds(lo + si * _ZS, _ZS)])
    plsc.subcore_barrier()
    return scarry

  lax.fori_loop(0, _S, sweep, 0)


def kernel(updates, mask):
  m = mask.astype(jnp.int32).reshape(-1)
  u = updates.reshape(-1)
  out = _unpool(m, u)
  return out.reshape(_B, _HO, _WO, _C)
